# Initial kernel scaffold; baseline (speedup 1.0000x reference)
import jax
import jax.numpy as jnp
from jax.experimental import pallas as pl


def kernel(x, edge_index, W1l, b1l, W1r, W2l, b2l, W2r):
    raise NotImplementedError("write your pallas kernel here")


# speedup vs baseline: 6.2581x; 6.2581x over previous
import functools

import jax
import jax.numpy as jnp
from jax import lax
from jax.experimental import pallas as pl
from jax.experimental.pallas import tpu as pltpu
from jax.experimental.pallas import tpu_sc as plsc

NC = 2
NS = 16
NW = NC * NS
CH = 128
ZR = 80
CW = 2000


@functools.cache
def _seg_sum(n, e, d, with_count):
    assert e % CH == 0 and n % ZR == 0 and d % 16 == 0
    nchunk = e // CH
    base_trips = nchunk // NW
    rem = nchunk % NW
    n_units = n // ZR
    units_per_sub = -(-n_units // NS)
    ncw = n // CW

    mesh = plsc.VectorSubcoreMesh(core_axis_name="c", subcore_axis_name="s")
    out_type = [jax.ShapeDtypeStruct((NC * n, d), jnp.float32)]
    if with_count:
        out_type.append(jax.ShapeDtypeStruct((NC * n,), jnp.float32))

    scratch = [
        pltpu.VMEM_SHARED((n, d), jnp.float32),
        pltpu.VMEM((CH,), jnp.int32),
        pltpu.VMEM((CH,), jnp.int32),
        pltpu.VMEM((CH, d), jnp.float32),
        pltpu.VMEM((ZR, d), jnp.float32),
        pltpu.SemaphoreType.DMA,
    ]
    if with_count:
        scratch += [
            pltpu.VMEM_SHARED((n,), jnp.float32),
            pltpu.VMEM((CH,), jnp.float32),
            pltpu.VMEM((CW,), jnp.float32),
        ]

    def body(*refs):
        if with_count:
            (tab, srcr, dstr, out, cnt_out,
             acc, srcb, dstb, rowb, zbuf, sem, cacc, onesb, czbuf) = refs
        else:
            (tab, srcr, dstr, out,
             acc, srcb, dstb, rowb, zbuf, sem) = refs

        c = lax.axis_index("c")
        s = lax.axis_index("s")
        w = s * NC + c

        def zrow(i, carry):
            for k in range(d // 16):
                zbuf[i, pl.ds(k * 16, 16)] = jnp.zeros((16,), jnp.float32)
            return carry
        lax.fori_loop(0, ZR, zrow, 0)
        for k in range(units_per_sub):
            u = s + k * NS

            @pl.when(u < n_units)
            def _():
                pltpu.sync_copy(zbuf, acc.at[pl.ds(u * ZR, ZR)])
        if with_count:
            def zc(i, carry):
                czbuf[pl.ds(i * 16, 16)] = jnp.zeros((16,), jnp.float32)
                return carry
            lax.fori_loop(0, CW // 16, zc, 0)

            @pl.when(s < ncw)
            def _():
                pltpu.sync_copy(czbuf, cacc.at[pl.ds(s * CW, CW)])
            for k in range(CH // 16):
                onesb[pl.ds(k * 16, 16)] = jnp.full((16,), 1.0, jnp.float32)
        plsc.subcore_barrier()

        trips = base_trips + jnp.where(w < rem, 1, 0)

        def chunk(i, carry):
            base = (w + i * NW) * CH
            pltpu.sync_copy(srcr.at[pl.ds(base, CH)], srcb)
            pltpu.sync_copy(dstr.at[pl.ds(base, CH)], dstb)
            pltpu.async_copy(tab.at[srcb], rowb, sem).wait()
            pltpu.sync_copy(rowb, acc.at[dstb], add=True)
            if with_count:
                pltpu.sync_copy(onesb, cacc.at[dstb], add=True)
            return carry

        lax.fori_loop(0, trips, chunk, 0)
        plsc.subcore_barrier()

        for k in range(units_per_sub):
            u = s + k * NS

            @pl.when(u < n_units)
            def _():
                pltpu.sync_copy(acc.at[pl.ds(u * ZR, ZR)], zbuf)
                pltpu.sync_copy(zbuf, out.at[pl.ds(c * n + u * ZR, ZR)])
        if with_count:
            @pl.when(s < ncw)
            def _():
                pltpu.sync_copy(cacc.at[pl.ds(s * CW, CW)], czbuf)
                pltpu.sync_copy(czbuf, cnt_out.at[pl.ds(c * n + s * CW, CW)])

    return pl.kernel(body, out_type=out_type, mesh=mesh,
                     scratch_types=scratch)


def _lin2_body(x_ref, wl_ref, wr_ref, b_ref, t_ref, r_ref):
    xv = x_ref[...]
    t_ref[...] = jnp.dot(xv, wl_ref[...], preferred_element_type=jnp.float32)
    r_ref[...] = (jnp.dot(xv, wr_ref[...], preferred_element_type=jnp.float32)
                  + b_ref[...])


def _mid_body(p_ref, cp_ref, r1_ref, w2r_ref, b2_ref, h_ref, r2_ref):
    cnt = jnp.maximum(cp_ref[0, :] + cp_ref[1, :], 1.0)
    agg = (p_ref[0] + p_ref[1]) / cnt[:, None]
    h = jnp.maximum(agg + r1_ref[...], 0.0)
    h_ref[...] = h
    r2_ref[...] = (jnp.dot(h, w2r_ref[...], preferred_element_type=jnp.float32)
                   + b2_ref[...])


def _out_body(q_ref, cp_ref, r2_ref, w2l_ref, o_ref):
    cnt = jnp.maximum(cp_ref[0, :] + cp_ref[1, :], 1.0)
    agg = (q_ref[0] + q_ref[1]) / cnt[:, None]
    o_ref[...] = (jnp.dot(agg, w2l_ref[...], preferred_element_type=jnp.float32)
                  + r2_ref[...])


def kernel(x, edge_index, W1l, b1l, W1r, W2l, b2l, W2r):
    n, f = x.shape
    e = edge_index.shape[1]
    hid = W1l.shape[0]
    ncls = W2l.shape[0]
    src = edge_index[0]
    dst = edge_index[1]

    f32 = jnp.float32
    t1, r1 = pl.pallas_call(
        _lin2_body,
        out_shape=[jax.ShapeDtypeStruct((n, hid), f32),
                   jax.ShapeDtypeStruct((n, hid), f32)],
    )(x, W1l.T, W1r.T, b1l[None, :])

    p, cnt2 = _seg_sum(n, e, hid, True)(t1, src, dst)
    p = p.reshape(2, n, hid)
    cp = cnt2.reshape(2, n)

    h, r2 = pl.pallas_call(
        _mid_body,
        out_shape=[jax.ShapeDtypeStruct((n, hid), f32),
                   jax.ShapeDtypeStruct((n, ncls), f32)],
    )(p, cp, r1, W2r.T, b2l[None, :])

    (q,) = _seg_sum(n, e, hid, False)(h, src, dst)
    q = q.reshape(2, n, hid)

    out = pl.pallas_call(
        _out_body,
        out_shape=jax.ShapeDtypeStruct((n, ncls), f32),
    )(q, cp, r2, W2l.T)
    return out
